# baseline (speedup 1.0000x reference)
import functools

import jax
import jax.numpy as jnp
from jax import lax
from jax.experimental import pallas as pl
from jax.experimental.pallas import tpu as pltpu
from jax.experimental.pallas import tpu_sc as plsc

B = 16384
D = 32
CHUNK = 128
NW = 32
BPW = B // NW
NQ = BPW // 4
LANES = 16


def _body(users_ref, games_ref, ue_ref, ge_ref, ub_ref, gb_ref, w_ref,
          out_ref, uflat, gflat, ubuf, gbuf, ubv, gbv, wv, outv, sem, semb2,
          semb):
  wid = lax.axis_index("s") * 2 + lax.axis_index("c")
  base = wid * BPW
  pltpu.sync_copy(users_ref.at[pl.ds(base, BPW)], uflat.at[pl.ds(0, BPW)])
  pltpu.sync_copy(games_ref.at[pl.ds(base, BPW)], gflat.at[pl.ds(0, BPW)])
  pltpu.sync_copy(w_ref, wv)
  zeros = jnp.zeros((LANES,), jnp.int32)
  uflat[pl.ds(BPW, LANES)] = zeros
  gflat[pl.ds(BPW, LANES)] = zeros

  bias_copies = []
  for j in range(BPW // CHUNK):
    sl = pl.ds(j * CHUNK, CHUNK)
    bias_copies.append(
        pltpu.async_copy(ub_ref.at[uflat.at[sl]], ubv.at[sl], semb))
    bias_copies.append(
        pltpu.async_copy(gb_ref.at[gflat.at[sl]], gbv.at[sl], semb))
  for c in bias_copies:
    c.wait()

  iota = lax.iota(jnp.int32, LANES)
  wulo = wv[pl.ds(0, 16)]
  wuhi = wv[pl.ds(16, 16)]
  wglo = wv[pl.ds(32, 16)]
  wghi = wv[pl.ds(48, 16)]
  fcb = wv[pl.ds(64, 16)][0]

  sems = (sem, semb2)

  def fire(q, slot):
    uvec = uflat[pl.ds(q * 4, 16)]
    gvec = gflat[pl.ds(q * 4, 16)]
    for l in range(4):
      u = uvec[l]
      g = gvec[l]
      cu = pl.multiple_of(lax.shift_right_logical(u, 7) * CHUNK, CHUNK)
      cg = pl.multiple_of(lax.shift_right_logical(g, 7) * CHUNK, CHUNK)
      pltpu.async_copy(ue_ref.at[:, pl.ds(cu, CHUNK)], ubuf.at[slot, l],
                       sems[slot])
      pltpu.async_copy(ge_ref.at[:, pl.ds(cg, CHUNK)], gbuf.at[slot, l],
                       sems[slot])

  def drain(slot):
    for _ in range(8):
      pltpu.make_async_copy(
          ue_ref.at[:, pl.ds(0, CHUNK)], ubuf.at[0, 0], sems[slot]).wait()

  def compute(q, slot, acc):
    uvec = uflat[pl.ds(q * 4, 16)]
    gvec = gflat[pl.ds(q * 4, 16)]
    for l in range(4):
      lu = uvec[l] & (CHUNK - 1)
      lg = gvec[l] & (CHUNK - 1)
      luv = jnp.zeros((LANES,), jnp.int32) + lu
      lgv = jnp.zeros((LANES,), jnp.int32) + lg
      vlo = plsc.load_gather(ubuf.at[slot, l], [iota, luv])
      vhi = plsc.load_gather(ubuf.at[slot, l], [iota + 16, luv])
      glo = plsc.load_gather(gbuf.at[slot, l], [iota, lgv])
      ghi = plsc.load_gather(gbuf.at[slot, l], [iota + 16, lgv])
      p = vlo * wulo + vhi * wuhi + glo * wglo + ghi * wghi
      s = plsc.cumsum(p)
      dot = s[15]
      lane = (q & 3) * 4 + l
      acc = jnp.where(iota == lane, dot, acc)

    @pl.when((q & 3) == 3)
    def _():
      sl = pl.ds((q - 3) * 4, 16)
      outv[sl] = acc + ubv[sl] + gbv[sl] + fcb

    return jnp.where((q & 3) == 3, jnp.zeros((LANES,), jnp.float32), acc)

  fire(0, 0)

  def pair(p, acc):
    q0 = p * 2
    fire(q0 + 1, 1)
    drain(0)
    acc = compute(q0, 0, acc)
    fire(q0 + 2, 0)
    drain(1)
    acc = compute(q0 + 1, 1, acc)
    return acc

  lax.fori_loop(0, NQ // 2, pair, jnp.zeros((LANES,), jnp.float32))
  drain(0)

  pltpu.sync_copy(outv, out_ref.at[pl.ds(base, BPW)])


def kernel(users, games, user_embed, game_embed, user_bias, game_bias,
           fc_w, fc_b):
  users1d = users.astype(jnp.int32).reshape(-1)
  games1d = games.astype(jnp.int32).reshape(-1)
  ue_t = user_embed.T
  ge_t = game_embed.T
  ub_flat = user_bias.reshape(-1)
  gb_flat = game_bias.reshape(-1)
  wlin = jnp.concatenate(
      [fc_w.reshape(-1), fc_b.reshape(-1),
       jnp.zeros((63,), jnp.float32)])

  run = functools.partial(
      pl.kernel,
      out_type=jax.ShapeDtypeStruct((B,), jnp.float32),
      mesh=plsc.VectorSubcoreMesh(core_axis_name="c", subcore_axis_name="s"),
      compiler_params=pltpu.CompilerParams(
          needs_layout_passes=False, use_tc_tiling_on_sc=True),
      scratch_types=[
          pltpu.VMEM((BPW + LANES,), jnp.int32),
          pltpu.VMEM((BPW + LANES,), jnp.int32),
          pltpu.VMEM((2, 4, D, CHUNK), jnp.float32),
          pltpu.VMEM((2, 4, D, CHUNK), jnp.float32),
          pltpu.VMEM((BPW,), jnp.float32),
          pltpu.VMEM((BPW,), jnp.float32),
          pltpu.VMEM((CHUNK,), jnp.float32),
          pltpu.VMEM((BPW,), jnp.float32),
          pltpu.SemaphoreType.DMA,
          pltpu.SemaphoreType.DMA,
          pltpu.SemaphoreType.DMA,
      ],
  )(_body)

  out = run(users1d, games1d, ue_t, ge_t, ub_flat, gb_flat, wlin)
  return out.reshape(B, 1)


# speedup vs baseline: 1.0195x; 1.0195x over previous
import functools

import jax
import jax.numpy as jnp
from jax import lax
from jax.experimental import pallas as pl
from jax.experimental.pallas import tpu as pltpu
from jax.experimental.pallas import tpu_sc as plsc

B = 16384
D = 32
CHUNK = 128
NW = 32
BPW = B // NW
LANES = 16
NGP = 100096
GW = NGP // 2
GSH = 6272
GCH = 128
NGC = GSH // GCH
GR = 4


def _body(users_ref, games_ref, ue_ref, ge_ref, ub_ref, gb_ref, w_ref,
          out_ref, uflat, gflat, ubuf, pb, gfeatw, gidxw, ubv, gbv,
          wv, outv, spg, sem0, sem1, semb, semg, semsp):
  cid = lax.axis_index("c")
  sid = lax.axis_index("s")
  wid = sid * 2 + cid
  base = wid * BPW
  pltpu.sync_copy(users_ref.at[pl.ds(base, BPW)], uflat.at[pl.ds(0, BPW)])
  pltpu.sync_copy(games_ref.at[pl.ds(base, BPW)], gflat.at[pl.ds(0, BPW)])
  pltpu.sync_copy(w_ref, wv)
  zeros = jnp.zeros((LANES,), jnp.int32)
  uflat[pl.ds(BPW, LANES)] = zeros
  gflat[pl.ds(BPW, LANES)] = zeros

  bias_copies = []
  for j in range(BPW // CHUNK):
    sl = pl.ds(j * CHUNK, CHUNK)
    bias_copies.append(
        pltpu.async_copy(ub_ref.at[uflat.at[sl]], ubv.at[sl], semb))
    bias_copies.append(
        pltpu.async_copy(gb_ref.at[gflat.at[sl]], gbv.at[sl], semb))

  gstart = sid * GSH
  stage = ubuf.at[0, 0]

  def build(c, carry):
    co = jnp.minimum(gstart + c * GCH, NGP - GCH)
    co = pl.multiple_of(co * 1, CHUNK)
    pltpu.sync_copy(ge_ref.at[:, pl.ds(co, GCH)], stage)
    for f in range(D):
      for j in range(GCH // 32):
        v0 = stage[f, pl.ds(j * 32, 16)]
        v1 = stage[f, pl.ds(j * 32 + 16, 16)]
        w = plsc.bitcast(
            plsc.pack(v0, v1, format=plsc.PackFormat.INTERLEAVED),
            jnp.int32)
        pb[pl.ds(f * (GCH // 2) + j * 16, 16)] = w
    w0 = pl.multiple_of(lax.shift_right_logical(co, 1), 8)
    spc = []
    for f in range(D):
      spc.append(pltpu.async_copy(
          pb.at[pl.ds(f * (GCH // 2), GCH // 2)],
          spg.at[pl.ds(pl.multiple_of(f * GW + w0, 8), GCH // 2)],
          semsp))
    for s in spc:
      s.wait()
    return carry

  lax.fori_loop(0, NGC, build, 0)
  plsc.subcore_barrier()

  iota = lax.iota(jnp.int32, LANES)
  for ch in range(BPW // CHUNK):
    for j in range(CHUNK // 16):
      gv = gflat[pl.ds(ch * CHUNK + j * 16, 16)]
      bw = lax.shift_left(lax.shift_right_logical(gv, 5), 4) + (gv & 15)
      gidxw[ch, pl.ds(j * 16, 16)] = bw

  wulo = wv[pl.ds(0, 16)]
  wuhi = wv[pl.ds(16, 16)]
  wglo = wv[pl.ds(32, 16)]
  wghi = wv[pl.ds(48, 16)]
  fcb = wv[pl.ds(64, 16)][0]
  sems = (sem0, sem1)

  def fire(slot, uvec, lb):
    for l in range(2):
      u = uvec[lb + l]
      cu = pl.multiple_of(lax.shift_right_logical(u, 7) * CHUNK, CHUNK)
      pltpu.async_copy(ue_ref.at[:, pl.ds(cu, CHUNK)], ubuf.at[slot, l],
                       sems[slot])

  def drain(slot):
    for _ in range(2):
      pltpu.make_async_copy(
          ue_ref.at[:, pl.ds(0, CHUNK)], ubuf.at[0, 0], sems[slot]).wait()

  for c in bias_copies:
    c.wait()
  fire(0, uflat[pl.ds(0, 16)], 0)

  def octet(o, acc):
    uv = uflat[pl.ds(o * 8, 16)]
    uv2 = uflat[pl.ds(o * 8 + 8, 16)]
    half = (o & 1) * 8
    for k in range(4):
      if k < 3:
        fire((k + 1) & 1, uv, 2 * k + 2)
      else:
        fire((k + 1) & 1, uv2, 0)
      drain(k & 1)
      for l in range(2):
        lu = uv[2 * k + l] & (CHUNK - 1)
        luv = jnp.zeros((LANES,), jnp.int32) + lu
        vlo = plsc.load_gather(ubuf.at[k & 1, l], [iota, luv])
        vhi = plsc.load_gather(ubuf.at[k & 1, l], [iota + 16, luv])
        p = vlo * wulo + vhi * wuhi
        s = plsc.cumsum(p)
        dot = s[15]
        acc = jnp.where(iota == half + 2 * k + l, dot, acc)

    @pl.when((o & 1) == 1)
    def _():
      sl = pl.ds((o - 1) * 8, 16)
      outv[sl] = acc + ubv[sl] + gbv[sl] + fcb

    return jnp.where((o & 1) == 1, jnp.zeros((LANES,), jnp.float32), acc)

  lax.fori_loop(0, BPW // 8, octet, jnp.zeros((LANES,), jnp.float32))
  drain(0)

  for r in range(GR):
    copies = []
    for f in range(D):
      copies.append(pltpu.async_copy(
          spg.at[pl.ds(f * GW, GW)].at[gidxw.at[r]],
          gfeatw.at[pl.ds(f * CHUNK, CHUNK)], semg))
    for c in copies:
      c.wait()

    def game_group(g, carry, r=r):
      sl16 = pl.ds(r * CHUNK + g * 16, 16)
      gv = gflat[sl16]
      hv = lax.shift_right_logical(gv, 4) & 1
      gacc = jnp.zeros((LANES,), jnp.float32)
      for f in range(D):
        words = gfeatw[pl.ds(f * CHUNK + g * 16, 16)]
        a, b = plsc.unpack(plsc.bitcast(words, jnp.bfloat16),
                           format=plsc.PackFormat.INTERLEAVED)
        val = jnp.where(hv == 1, b, a)
        wf = jnp.zeros((LANES,), jnp.float32) + (
            wglo[f] if f < 16 else wghi[f - 16])
        gacc = gacc + val * wf
      outv[sl16] = outv[sl16] + gacc
      return carry

    lax.fori_loop(0, CHUNK // 16, game_group, 0)

  pltpu.sync_copy(outv, out_ref.at[pl.ds(base, BPW)])


def kernel(users, games, user_embed, game_embed, user_bias, game_bias,
           fc_w, fc_b):
  users1d = users.astype(jnp.int32).reshape(-1)
  games1d = games.astype(jnp.int32).reshape(-1)
  ue_t = user_embed.T
  ge_t = game_embed.T
  ub_flat = user_bias.reshape(-1)
  gb_flat = game_bias.reshape(-1)
  wlin = jnp.concatenate(
      [fc_w.reshape(-1), fc_b.reshape(-1),
       jnp.zeros((63,), jnp.float32)])

  run = functools.partial(
      pl.kernel,
      out_type=jax.ShapeDtypeStruct((B,), jnp.float32),
      mesh=plsc.VectorSubcoreMesh(core_axis_name="c", subcore_axis_name="s"),
      compiler_params=pltpu.CompilerParams(
          needs_layout_passes=False, use_tc_tiling_on_sc=True),
      scratch_types=[
          pltpu.VMEM((BPW + LANES,), jnp.int32),
          pltpu.VMEM((BPW + LANES,), jnp.int32),
          pltpu.VMEM((2, 2, D, CHUNK), jnp.float32),
          pltpu.VMEM((D * GCH // 2,), jnp.int32),
          pltpu.VMEM((D * CHUNK,), jnp.int32),
          pltpu.VMEM((BPW // CHUNK, CHUNK), jnp.int32),
          pltpu.VMEM((BPW,), jnp.float32),
          pltpu.VMEM((BPW,), jnp.float32),
          pltpu.VMEM((CHUNK,), jnp.float32),
          pltpu.VMEM((BPW,), jnp.float32),
          pltpu.VMEM_SHARED((D * GW,), jnp.int32),
          pltpu.SemaphoreType.DMA,
          pltpu.SemaphoreType.DMA,
          pltpu.SemaphoreType.DMA,
          pltpu.SemaphoreType.DMA,
          pltpu.SemaphoreType.DMA,
      ],
  )(_body)

  out = run(users1d, games1d, ue_t, ge_t, ub_flat, gb_flat, wlin)
  return out.reshape(B, 1)


# speedup vs baseline: 1.0792x; 1.0585x over previous
import functools

import jax
import jax.numpy as jnp
from jax import lax
from jax.experimental import pallas as pl
from jax.experimental.pallas import tpu as pltpu
from jax.experimental.pallas import tpu_sc as plsc

B = 16384
D = 32
CHUNK = 128
NW = 32
BPW = B // NW
LANES = 16
NGP = 100096
GW = NGP // 2
GSH = 6272
GCH = 128
NGC = GSH // GCH
GR = 4


def _body(users_ref, games_ref, ue_ref, ge_ref, ub_ref, gb_ref, w_ref,
          out_ref, uflat, gflat, ubuf, pb, gfeatw, gidxw, ubv, gbv,
          wv, outv, spg, sem0, sem1, sem2, sem3, semb, semg, semsp):
  cid = lax.axis_index("c")
  sid = lax.axis_index("s")
  wid = sid * 2 + cid
  base = wid * BPW
  pltpu.sync_copy(users_ref.at[pl.ds(base, BPW)], uflat.at[pl.ds(0, BPW)])
  pltpu.sync_copy(games_ref.at[pl.ds(base, BPW)], gflat.at[pl.ds(0, BPW)])
  pltpu.sync_copy(w_ref, wv)
  zeros = jnp.zeros((LANES,), jnp.int32)
  uflat[pl.ds(BPW, LANES)] = zeros
  gflat[pl.ds(BPW, LANES)] = zeros

  bias_copies = []
  for j in range(BPW // CHUNK):
    sl = pl.ds(j * CHUNK, CHUNK)
    bias_copies.append(
        pltpu.async_copy(ub_ref.at[uflat.at[sl]], ubv.at[sl], semb))
    bias_copies.append(
        pltpu.async_copy(gb_ref.at[gflat.at[sl]], gbv.at[sl], semb))

  gstart = sid * GSH
  stage = ubuf.at[0]

  def build(c, carry):
    co = jnp.minimum(gstart + c * GCH, NGP - GCH)
    co = pl.multiple_of(co * 1, CHUNK)
    pltpu.sync_copy(ge_ref.at[:, pl.ds(co, GCH)], stage)
    for f in range(D):
      for j in range(GCH // 32):
        v0 = stage[f, pl.ds(j * 32, 16)]
        v1 = stage[f, pl.ds(j * 32 + 16, 16)]
        w = plsc.bitcast(
            plsc.pack(v0, v1, format=plsc.PackFormat.INTERLEAVED),
            jnp.int32)
        pb[pl.ds(f * (GCH // 2) + j * 16, 16)] = w
    w0 = pl.multiple_of(lax.shift_right_logical(co, 1), 8)
    spc = []
    for f in range(D):
      spc.append(pltpu.async_copy(
          pb.at[pl.ds(f * (GCH // 2), GCH // 2)],
          spg.at[pl.ds(pl.multiple_of(f * GW + w0, 8), GCH // 2)],
          semsp))
    for s in spc:
      s.wait()
    return carry

  lax.fori_loop(0, NGC, build, 0)
  plsc.subcore_barrier()

  iota = lax.iota(jnp.int32, LANES)
  for ch in range(BPW // CHUNK):
    for j in range(CHUNK // 16):
      gv = gflat[pl.ds(ch * CHUNK + j * 16, 16)]
      bw = lax.shift_left(lax.shift_right_logical(gv, 5), 4) + (gv & 15)
      gidxw[ch, pl.ds(j * 16, 16)] = bw

  wulo = wv[pl.ds(0, 16)]
  wuhi = wv[pl.ds(16, 16)]
  wglo = wv[pl.ds(32, 16)]
  wghi = wv[pl.ds(48, 16)]
  fcb = wv[pl.ds(64, 16)][0]
  sems = (sem0, sem1, sem2, sem3)

  def fire1(slot, u):
    cu = pl.multiple_of(lax.shift_right_logical(u, 7) * CHUNK, CHUNK)
    pltpu.async_copy(ue_ref.at[:, pl.ds(cu, CHUNK)], ubuf.at[slot],
                     sems[slot])

  def drain1(slot):
    pltpu.make_async_copy(
        ue_ref.at[:, pl.ds(0, CHUNK)], ubuf.at[0], sems[slot]).wait()

  for c in bias_copies:
    c.wait()
  uv0 = uflat[pl.ds(0, 16)]
  for l in range(3):
    fire1(l, uv0[l])

  def octet(o, acc):
    uv = uflat[pl.ds(o * 8, 16)]
    uv2 = uflat[pl.ds(o * 8 + 8, 16)]
    half = (o & 1) * 8
    for l in range(8):
      slot = l % 4
      nxt = uv[l + 3] if l < 5 else uv2[l - 5]
      fire1((l + 3) % 4, nxt)
      drain1(slot)
      lu = uv[l] & (CHUNK - 1)
      luv = jnp.zeros((LANES,), jnp.int32) + lu
      vlo = plsc.load_gather(ubuf.at[slot], [iota, luv])
      vhi = plsc.load_gather(ubuf.at[slot], [iota + 16, luv])
      p = vlo * wulo + vhi * wuhi
      s = plsc.cumsum(p)
      dot = s[15]
      acc = jnp.where(iota == half + l, dot, acc)

    @pl.when((o & 1) == 1)
    def _():
      sl = pl.ds((o - 1) * 8, 16)
      outv[sl] = acc + ubv[sl] + gbv[sl] + fcb

    return jnp.where((o & 1) == 1, jnp.zeros((LANES,), jnp.float32), acc)

  lax.fori_loop(0, BPW // 8, octet, jnp.zeros((LANES,), jnp.float32))
  for l in range(3):
    drain1(l)

  for r in range(GR):
    copies = []
    for f in range(D):
      copies.append(pltpu.async_copy(
          spg.at[pl.ds(f * GW, GW)].at[gidxw.at[r]],
          gfeatw.at[pl.ds(f * CHUNK, CHUNK)], semg))
    for c in copies:
      c.wait()

    def game_group(g, carry, r=r):
      sl16 = pl.ds(r * CHUNK + g * 16, 16)
      gv = gflat[sl16]
      hv = lax.shift_right_logical(gv, 4) & 1
      gacc = jnp.zeros((LANES,), jnp.float32)
      for f in range(D):
        words = gfeatw[pl.ds(f * CHUNK + g * 16, 16)]
        a, b = plsc.unpack(plsc.bitcast(words, jnp.bfloat16),
                           format=plsc.PackFormat.INTERLEAVED)
        val = jnp.where(hv == 1, b, a)
        wf = jnp.zeros((LANES,), jnp.float32) + (
            wglo[f] if f < 16 else wghi[f - 16])
        gacc = gacc + val * wf
      outv[sl16] = outv[sl16] + gacc
      return carry

    lax.fori_loop(0, CHUNK // 16, game_group, 0)

  pltpu.sync_copy(outv, out_ref.at[pl.ds(base, BPW)])


def kernel(users, games, user_embed, game_embed, user_bias, game_bias,
           fc_w, fc_b):
  users1d = users.astype(jnp.int32).reshape(-1)
  games1d = games.astype(jnp.int32).reshape(-1)
  ue_t = user_embed.T
  ge_t = game_embed.T
  ub_flat = user_bias.reshape(-1)
  gb_flat = game_bias.reshape(-1)
  wlin = jnp.concatenate(
      [fc_w.reshape(-1), fc_b.reshape(-1),
       jnp.zeros((63,), jnp.float32)])

  run = functools.partial(
      pl.kernel,
      out_type=jax.ShapeDtypeStruct((B,), jnp.float32),
      mesh=plsc.VectorSubcoreMesh(core_axis_name="c", subcore_axis_name="s"),
      compiler_params=pltpu.CompilerParams(
          needs_layout_passes=False, use_tc_tiling_on_sc=True),
      scratch_types=[
          pltpu.VMEM((BPW + LANES,), jnp.int32),
          pltpu.VMEM((BPW + LANES,), jnp.int32),
          pltpu.VMEM((4, D, CHUNK), jnp.float32),
          pltpu.VMEM((D * GCH // 2,), jnp.int32),
          pltpu.VMEM((D * CHUNK,), jnp.int32),
          pltpu.VMEM((BPW // CHUNK, CHUNK), jnp.int32),
          pltpu.VMEM((BPW,), jnp.float32),
          pltpu.VMEM((BPW,), jnp.float32),
          pltpu.VMEM((CHUNK,), jnp.float32),
          pltpu.VMEM((BPW,), jnp.float32),
          pltpu.VMEM_SHARED((D * GW,), jnp.int32),
          pltpu.SemaphoreType.DMA,
          pltpu.SemaphoreType.DMA,
          pltpu.SemaphoreType.DMA,
          pltpu.SemaphoreType.DMA,
          pltpu.SemaphoreType.DMA,
          pltpu.SemaphoreType.DMA,
          pltpu.SemaphoreType.DMA,
      ],
  )(_body)

  out = run(users1d, games1d, ue_t, ge_t, ub_flat, gb_flat, wlin)
  return out.reshape(B, 1)


# speedup vs baseline: 1.1964x; 1.1087x over previous
import functools

import jax
import jax.numpy as jnp
from jax import lax
from jax.experimental import pallas as pl
from jax.experimental.pallas import tpu as pltpu
from jax.experimental.pallas import tpu_sc as plsc

B = 16384
D = 32
CHUNK = 128
NW = 32
BPW = B // NW
LANES = 16
NGP = 100096
GW = NGP // 2
GSH = 6272
GCH = 128
NGC = GSH // GCH
GR = 4


def _body(users_ref, games_ref, ue_ref, ge_ref, ub_ref, gb_ref, w_ref,
          out_ref, uflat, gflat, ubuf, pb, gfeatw, gidxw, ubv, gbv,
          wv, outv, spg, sem0, sem1, sem2, sem3, semb, semg, semsp):
  cid = lax.axis_index("c")
  sid = lax.axis_index("s")
  wid = sid * 2 + cid
  base = wid * BPW
  pltpu.sync_copy(users_ref.at[pl.ds(base, BPW)], uflat.at[pl.ds(0, BPW)])
  pltpu.sync_copy(games_ref.at[pl.ds(base, BPW)], gflat.at[pl.ds(0, BPW)])
  pltpu.sync_copy(w_ref, wv)
  zeros = jnp.zeros((LANES,), jnp.int32)
  uflat[pl.ds(BPW, LANES)] = zeros
  gflat[pl.ds(BPW, LANES)] = zeros

  bias_copies = []
  for j in range(BPW // CHUNK):
    sl = pl.ds(j * CHUNK, CHUNK)
    bias_copies.append(
        pltpu.async_copy(ub_ref.at[uflat.at[sl]], ubv.at[sl], semb))
    bias_copies.append(
        pltpu.async_copy(gb_ref.at[gflat.at[sl]], gbv.at[sl], semb))

  gstart = sid * GSH

  def chunk_off(c):
    co = jnp.minimum(gstart + c * GCH, NGP - GCH)
    return pl.multiple_of(co * 1, CHUNK)

  stsems = (sem0, sem1)

  def stage_fire(c, slot):
    pltpu.async_copy(ge_ref.at[:, pl.ds(chunk_off(c), GCH)],
                     ubuf.at[slot], stsems[slot])

  def stage_wait(slot):
    pltpu.make_async_copy(
        ge_ref.at[:, pl.ds(0, GCH)], ubuf.at[0], stsems[slot]).wait()

  def pack_chunk(c, slot):
    co = chunk_off(c)
    for f in range(D):
      for j in range(GCH // 32):
        v0 = ubuf[slot, f, pl.ds(j * 32, 16)]
        v1 = ubuf[slot, f, pl.ds(j * 32 + 16, 16)]
        w = plsc.bitcast(
            plsc.pack(v0, v1, format=plsc.PackFormat.INTERLEAVED),
            jnp.int32)
        pb[pl.ds(f * (GCH // 2) + j * 16, 16)] = w
    w0 = pl.multiple_of(lax.shift_right_logical(co, 1), 8)
    spc = []
    for f in range(D):
      spc.append(pltpu.async_copy(
          pb.at[pl.ds(f * (GCH // 2), GCH // 2)],
          spg.at[pl.ds(pl.multiple_of(f * GW + w0, 8), GCH // 2)],
          semsp))
    for s in spc:
      s.wait()

  stage_fire(0, 0)

  def build_pair(p, carry):
    c0 = p * 2
    stage_fire(c0 + 1, 1)
    stage_wait(0)
    pack_chunk(c0, 0)
    stage_fire(c0 + 2, 0)
    stage_wait(1)
    pack_chunk(c0 + 1, 1)
    return carry

  lax.fori_loop(0, NGC // 2, build_pair, 0)
  stage_wait(0)
  pack_chunk(NGC - 1, 0)
  plsc.subcore_barrier()

  iota = lax.iota(jnp.int32, LANES)
  for ch in range(BPW // CHUNK):
    for j in range(CHUNK // 16):
      gv = gflat[pl.ds(ch * CHUNK + j * 16, 16)]
      bw = lax.shift_left(lax.shift_right_logical(gv, 5), 4) + (gv & 15)
      gidxw[ch, pl.ds(j * 16, 16)] = bw

  wulo = wv[pl.ds(0, 16)]
  wuhi = wv[pl.ds(16, 16)]
  wglo = wv[pl.ds(32, 16)]
  wghi = wv[pl.ds(48, 16)]
  fcb = wv[pl.ds(64, 16)][0]
  sems = (sem0, sem1, sem2, sem3)

  def fire1(slot, u):
    cu = pl.multiple_of(lax.shift_right_logical(u, 7) * CHUNK, CHUNK)
    pltpu.async_copy(ue_ref.at[:, pl.ds(cu, CHUNK)], ubuf.at[slot],
                     sems[slot])

  def drain1(slot):
    pltpu.make_async_copy(
        ue_ref.at[:, pl.ds(0, CHUNK)], ubuf.at[0], sems[slot]).wait()

  for c in bias_copies:
    c.wait()
  uv0 = uflat[pl.ds(0, 16)]
  for l in range(3):
    fire1(l, uv0[l])

  def octet(o, acc):
    uv = uflat[pl.ds(o * 8, 16)]
    uv2 = uflat[pl.ds(o * 8 + 8, 16)]
    half = (o & 1) * 8
    for l in range(8):
      slot = l % 4
      nxt = uv[l + 3] if l < 5 else uv2[l - 5]
      fire1((l + 3) % 4, nxt)
      drain1(slot)
      lu = uv[l] & (CHUNK - 1)
      luv = jnp.zeros((LANES,), jnp.int32) + lu
      vlo = plsc.load_gather(ubuf.at[slot], [iota, luv])
      vhi = plsc.load_gather(ubuf.at[slot], [iota + 16, luv])
      p = vlo * wulo + vhi * wuhi
      s = plsc.cumsum(p)
      dot = s[15]
      acc = jnp.where(iota == half + l, dot, acc)

    @pl.when((o & 1) == 1)
    def _():
      sl = pl.ds((o - 1) * 8, 16)
      outv[sl] = acc + ubv[sl] + gbv[sl] + fcb

    return jnp.where((o & 1) == 1, jnp.zeros((LANES,), jnp.float32), acc)

  lax.fori_loop(0, BPW // 8, octet, jnp.zeros((LANES,), jnp.float32))
  for l in range(3):
    drain1(l)

  for r in range(GR):
    copies = []
    for f in range(D):
      copies.append(pltpu.async_copy(
          spg.at[pl.ds(f * GW, GW)].at[gidxw.at[r]],
          gfeatw.at[pl.ds(f * CHUNK, CHUNK)], semg))
    for c in copies:
      c.wait()

    def game_group(g, carry, r=r):
      sl16 = pl.ds(r * CHUNK + g * 16, 16)
      gv = gflat[sl16]
      hv = lax.shift_right_logical(gv, 4) & 1
      gacc = jnp.zeros((LANES,), jnp.float32)
      for f in range(D):
        words = gfeatw[pl.ds(f * CHUNK + g * 16, 16)]
        a, b = plsc.unpack(plsc.bitcast(words, jnp.bfloat16),
                           format=plsc.PackFormat.INTERLEAVED)
        val = jnp.where(hv == 1, b, a)
        wf = jnp.zeros((LANES,), jnp.float32) + (
            wglo[f] if f < 16 else wghi[f - 16])
        gacc = gacc + val * wf
      outv[sl16] = outv[sl16] + gacc
      return carry

    lax.fori_loop(0, CHUNK // 16, game_group, 0)

  pltpu.sync_copy(outv, out_ref.at[pl.ds(base, BPW)])


def kernel(users, games, user_embed, game_embed, user_bias, game_bias,
           fc_w, fc_b):
  users1d = users.astype(jnp.int32).reshape(-1)
  games1d = games.astype(jnp.int32).reshape(-1)
  ue_t = user_embed.T
  ge_t = game_embed.T
  ub_flat = user_bias.reshape(-1)
  gb_flat = game_bias.reshape(-1)
  wlin = jnp.concatenate(
      [fc_w.reshape(-1), fc_b.reshape(-1),
       jnp.zeros((63,), jnp.float32)])

  run = functools.partial(
      pl.kernel,
      out_type=jax.ShapeDtypeStruct((B,), jnp.float32),
      mesh=plsc.VectorSubcoreMesh(core_axis_name="c", subcore_axis_name="s"),
      compiler_params=pltpu.CompilerParams(
          needs_layout_passes=False, use_tc_tiling_on_sc=True),
      scratch_types=[
          pltpu.VMEM((BPW + LANES,), jnp.int32),
          pltpu.VMEM((BPW + LANES,), jnp.int32),
          pltpu.VMEM((4, D, CHUNK), jnp.float32),
          pltpu.VMEM((D * GCH // 2,), jnp.int32),
          pltpu.VMEM((D * CHUNK,), jnp.int32),
          pltpu.VMEM((BPW // CHUNK, CHUNK), jnp.int32),
          pltpu.VMEM((BPW,), jnp.float32),
          pltpu.VMEM((BPW,), jnp.float32),
          pltpu.VMEM((CHUNK,), jnp.float32),
          pltpu.VMEM((BPW,), jnp.float32),
          pltpu.VMEM_SHARED((D * GW,), jnp.int32),
          pltpu.SemaphoreType.DMA,
          pltpu.SemaphoreType.DMA,
          pltpu.SemaphoreType.DMA,
          pltpu.SemaphoreType.DMA,
          pltpu.SemaphoreType.DMA,
          pltpu.SemaphoreType.DMA,
          pltpu.SemaphoreType.DMA,
      ],
  )(_body)

  out = run(users1d, games1d, ue_t, ge_t, ub_flat, gb_flat, wlin)
  return out.reshape(B, 1)
